# baseline (speedup 1.0000x reference)
import jax
import jax.numpy as jnp
from jax import lax
from jax.experimental import pallas as pl
from jax.experimental.pallas import tpu as pltpu
from jax.experimental.pallas import tpu_sc as plsc

_NUM_IN = 8192
_NUM_OUT = 2048
_BF = 4
_BATCH = 4096
_L = 16
_NC = 2
_NS = 16
_NW = _NC * _NS

_S = 768
_ROWS = _S // _NW
_R = 4
_NCHUNK = _ROWS // _R
_OG = _NUM_OUT // _L

_RB = 128
_NT = 16
_CS = _NUM_IN // _NT


def _sc_body(x_hbm, t_hbm, w_hbm, tw_hbm, out_hbm,
             x_tile, t_tile, w_tile, tw_tile, out_tile,
             xs0, xs1, os0, os1):
    wid = lax.axis_index("s") * _NC + lax.axis_index("c")
    base = wid * _ROWS
    pltpu.sync_copy(w_hbm, w_tile)
    pltpu.sync_copy(tw_hbm, tw_tile)
    pltpu.sync_copy(t_hbm.at[pl.ds(base, _ROWS)], t_tile)
    lane4 = lax.broadcasted_iota(jnp.int32, (_L,), 0) * _BF
    xsems = (xs0, xs1)
    osems = (os0, os1)

    def x_copy(ci, p):
        return pltpu.make_async_copy(
            x_hbm.at[pl.ds(base + ci * _R, _R)], x_tile.at[p], xsems[p])

    def o_copy(ci, p):
        return pltpu.make_async_copy(
            out_tile.at[p], out_hbm.at[pl.ds(base + ci * _R, _R)], osems[p])

    def compute(ci, p):
        orow = out_tile.at[p]
        tbs = [plsc.load_gather(t_tile,
                                [jnp.full((_L,), ci * _R + r, jnp.int32)]
                                ).astype(jnp.float32)
               for r in range(_R)]

        @plsc.parallel_loop(0, _OG, unroll=4)
        def _(g):
            o0 = g * _L
            cb = o0 * _BF
            tw_v = tw_tile[pl.ds(o0, _L)]
            wseg = w_tile.at[pl.ds(cb, _L * _BF)]
            w_vs = [plsc.load_gather(wseg, [lane4 + j]) for j in range(_BF)]
            for r in range(_R):
                seg = x_tile.at[p, r, pl.ds(cb, _L * _BF)]
                acc = tw_v * tbs[r]
                for j in range(_BF):
                    acc = acc + w_vs[j] * plsc.load_gather(seg, [lane4 + j])
                orow[r, pl.ds(o0, _L)] = acc

    x_copy(0, 0).start()

    def pair_body(k, carry):
        for p in range(2):
            ci = 2 * k + p

            @pl.when(ci + 1 < _NCHUNK)
            def _():
                x_copy(ci + 1, 1 - p).start()

            x_copy(ci, p).wait()

            @pl.when(ci >= 2)
            def _():
                o_copy(ci - 2, p).wait()

            compute(ci, p)
            o_copy(ci, p).start()
        return carry

    lax.fori_loop(0, _NCHUNK // 2, pair_body, 0)
    o_copy(_NCHUNK - 2, 0).wait()
    o_copy(_NCHUNK - 1, 1).wait()


def _tc_body(x_ref, t_ref, w_ref, tw_ref, o_ref):
    t_term = t_ref[...].astype(jnp.float32) * tw_ref[...]
    sel_in = lax.broadcasted_iota(jnp.int32, (_CS, _CS // _BF), 0)
    sel_out = lax.broadcasted_iota(jnp.int32, (_CS, _CS // _BF), 1)
    selm = (sel_in // _BF == sel_out).astype(jnp.float32)
    for tt in range(_NT):
        y = x_ref[:, tt * _CS:(tt + 1) * _CS] * w_ref[:, tt * _CS:(tt + 1) * _CS]
        s = jnp.dot(y, selm, preferred_element_type=jnp.float32)
        o_ref[:, tt * 128:(tt + 1) * 128] = s + t_term[:, tt * 128:(tt + 1) * 128]


def kernel(x, t, weight_vals, t_weights):
    tw = t_weights.reshape(_NUM_OUT)

    mesh = plsc.VectorSubcoreMesh(core_axis_name="c", subcore_axis_name="s")
    sc_out = pl.kernel(
        _sc_body,
        out_type=jax.ShapeDtypeStruct((_S, _NUM_OUT), jnp.float32),
        mesh=mesh,
        scratch_types=[
            pltpu.VMEM((2, _R, _NUM_IN), jnp.float32),
            pltpu.VMEM((_ROWS,), jnp.int32),
            pltpu.VMEM((_NUM_IN,), jnp.float32),
            pltpu.VMEM((_NUM_OUT,), jnp.float32),
            pltpu.VMEM((2, _R, _NUM_OUT), jnp.float32),
            pltpu.SemaphoreType.DMA,
            pltpu.SemaphoreType.DMA,
            pltpu.SemaphoreType.DMA,
            pltpu.SemaphoreType.DMA,
        ],
        compiler_params=pltpu.CompilerParams(needs_layout_passes=False),
    )(x, t, weight_vals, tw)

    nb = (_BATCH - _S) // _RB
    sb = _S // _RB
    tc_out = pl.pallas_call(
        _tc_body,
        grid=(nb,),
        in_specs=[
            pl.BlockSpec((_RB, _NUM_IN), lambda i: (i + sb, 0)),
            pl.BlockSpec((_RB, 1), lambda i: (i + sb, 0)),
            pl.BlockSpec((1, _NUM_IN), lambda i: (0, 0)),
            pl.BlockSpec((1, _NUM_OUT), lambda i: (0, 0)),
        ],
        out_specs=pl.BlockSpec((_RB, _NUM_OUT), lambda i: (i + sb, 0)),
        out_shape=jax.ShapeDtypeStruct((_BATCH, _NUM_OUT), jnp.float32),
    )(x, t.reshape(_BATCH, 1), weight_vals.reshape(1, _NUM_IN),
      tw.reshape(1, _NUM_OUT))

    return lax.dynamic_update_slice(tc_out, sc_out, (0, 0))


# speedup vs baseline: 1.0616x; 1.0616x over previous
import jax
import jax.numpy as jnp
from jax import lax
from jax.experimental import pallas as pl
from jax.experimental.pallas import tpu as pltpu
from jax.experimental.pallas import tpu_sc as plsc

_NUM_IN = 8192
_NUM_OUT = 2048
_BF = 4
_BATCH = 4096
_L = 16
_NC = 2
_NS = 16
_NW = _NC * _NS

_S = 512
_ROWS = _S // _NW
_R = 4
_NCHUNK = _ROWS // _R
_OG = _NUM_OUT // _L

_RB = 256
_NT = 16
_CS = _NUM_IN // _NT


def _sc_body(x_hbm, t_hbm, w_hbm, tw_hbm, out_hbm,
             x_tile, t_tile, w_tile, tw_tile, out_tile,
             xs0, xs1, os0, os1):
    wid = lax.axis_index("s") * _NC + lax.axis_index("c")
    base = wid * _ROWS
    pltpu.sync_copy(w_hbm, w_tile)
    pltpu.sync_copy(tw_hbm, tw_tile)
    pltpu.sync_copy(t_hbm.at[pl.ds(base, _ROWS)], t_tile)
    lane4 = lax.broadcasted_iota(jnp.int32, (_L,), 0) * _BF
    xsems = (xs0, xs1)
    osems = (os0, os1)

    def x_copy(ci, p):
        return pltpu.make_async_copy(
            x_hbm.at[pl.ds(base + ci * _R, _R)], x_tile.at[p], xsems[p])

    def o_copy(ci, p):
        return pltpu.make_async_copy(
            out_tile.at[p], out_hbm.at[pl.ds(base + ci * _R, _R)], osems[p])

    def compute(ci, p):
        orow = out_tile.at[p]
        tbs = [plsc.load_gather(t_tile,
                                [jnp.full((_L,), ci * _R + r, jnp.int32)]
                                ).astype(jnp.float32)
               for r in range(_R)]

        @plsc.parallel_loop(0, _OG, unroll=4)
        def _(g):
            o0 = g * _L
            cb = o0 * _BF
            tw_v = tw_tile[pl.ds(o0, _L)]
            wseg = w_tile.at[pl.ds(cb, _L * _BF)]
            w_vs = [plsc.load_gather(wseg, [lane4 + j]) for j in range(_BF)]
            for r in range(_R):
                seg = x_tile.at[p, r, pl.ds(cb, _L * _BF)]
                acc = tw_v * tbs[r]
                for j in range(_BF):
                    acc = acc + w_vs[j] * plsc.load_gather(seg, [lane4 + j])
                orow[r, pl.ds(o0, _L)] = acc

    x_copy(0, 0).start()

    def pair_body(k, carry):
        for p in range(2):
            ci = 2 * k + p

            @pl.when(ci + 1 < _NCHUNK)
            def _():
                x_copy(ci + 1, 1 - p).start()

            x_copy(ci, p).wait()

            @pl.when(ci >= 2)
            def _():
                o_copy(ci - 2, p).wait()

            compute(ci, p)
            o_copy(ci, p).start()
        return carry

    lax.fori_loop(0, _NCHUNK // 2, pair_body, 0)
    o_copy(_NCHUNK - 2, 0).wait()
    o_copy(_NCHUNK - 1, 1).wait()


def _tc_body(x_ref, t_ref, w_ref, tw_ref, o_ref):
    t_term = t_ref[...].astype(jnp.float32) * tw_ref[...]
    sel_in = lax.broadcasted_iota(jnp.int32, (_CS, _CS // _BF), 0)
    sel_out = lax.broadcasted_iota(jnp.int32, (_CS, _CS // _BF), 1)
    selm = (sel_in // _BF == sel_out).astype(jnp.float32)
    for tt in range(_NT):
        y = x_ref[:, tt * _CS:(tt + 1) * _CS] * w_ref[:, tt * _CS:(tt + 1) * _CS]
        s = jnp.dot(y, selm, preferred_element_type=jnp.float32)
        o_ref[:, tt * 128:(tt + 1) * 128] = s + t_term[:, tt * 128:(tt + 1) * 128]


def kernel(x, t, weight_vals, t_weights):
    tw = t_weights.reshape(_NUM_OUT)

    mesh = plsc.VectorSubcoreMesh(core_axis_name="c", subcore_axis_name="s")
    sc_out = pl.kernel(
        _sc_body,
        out_type=jax.ShapeDtypeStruct((_S, _NUM_OUT), jnp.float32),
        mesh=mesh,
        scratch_types=[
            pltpu.VMEM((2, _R, _NUM_IN), jnp.float32),
            pltpu.VMEM((_ROWS,), jnp.int32),
            pltpu.VMEM((_NUM_IN,), jnp.float32),
            pltpu.VMEM((_NUM_OUT,), jnp.float32),
            pltpu.VMEM((2, _R, _NUM_OUT), jnp.float32),
            pltpu.SemaphoreType.DMA,
            pltpu.SemaphoreType.DMA,
            pltpu.SemaphoreType.DMA,
            pltpu.SemaphoreType.DMA,
        ],
        compiler_params=pltpu.CompilerParams(needs_layout_passes=False),
    )(x, t, weight_vals, tw)

    nb = (_BATCH - _S) // _RB
    sb = _S // _RB
    tc_out = pl.pallas_call(
        _tc_body,
        grid=(nb,),
        in_specs=[
            pl.BlockSpec((_RB, _NUM_IN), lambda i: (i + sb, 0)),
            pl.BlockSpec((_RB, 1), lambda i: (i + sb, 0)),
            pl.BlockSpec((1, _NUM_IN), lambda i: (0, 0)),
            pl.BlockSpec((1, _NUM_OUT), lambda i: (0, 0)),
        ],
        out_specs=pl.BlockSpec((_RB, _NUM_OUT), lambda i: (i + sb, 0)),
        out_shape=jax.ShapeDtypeStruct((_BATCH, _NUM_OUT), jnp.float32),
    )(x, t.reshape(_BATCH, 1), weight_vals.reshape(1, _NUM_IN),
      tw.reshape(1, _NUM_OUT))

    return lax.dynamic_update_slice(tc_out, sc_out, (0, 0))
